# baseline (speedup 1.0000x reference)
import functools

import jax
import jax.numpy as jnp
from jax import lax
from jax.experimental import pallas as pl
from jax.experimental.pallas import tpu as pltpu
from jax.experimental.pallas import tpu_sc as plsc

_F = 26
_V = 100000
_D = 32
_B = 16384

_NC = 2
_NS = 16
_NW = _NC * _NS

_R = _F * _D
_RPW = _R // _NW
_VEC = 16
_UNROLL = 8
_NITER = _B // (_VEC * _UNROLL)


def _make_kernel(row0, nrows):
    rpw = nrows // _NW
    mesh = plsc.VectorSubcoreMesh(core_axis_name="c", subcore_axis_name="s")

    @functools.partial(
        pl.kernel,
        mesh=mesh,
        out_type=jax.ShapeDtypeStruct((nrows, _B), jnp.float32),
        compiler_params=pltpu.CompilerParams(
            use_tc_tiling_on_sc=False, needs_layout_passes=False),
        scratch_types=[
            pltpu.VMEM((_V,), jnp.float32),
            pltpu.VMEM((_B,), jnp.float32),
            pltpu.SemaphoreType.DMA,
            pltpu.SemaphoreType.DMA,
        ],
    )
    def emb(tv_hbm, x_hbm, out_hbm, row_v, buf_v, sem_r, sem_x):
        wid = lax.axis_index("s") * _NC + lax.axis_index("c")
        r0 = wid * rpw

        def row_body(rl, carry):
            r = r0 + rl
            f = (row0 + r) // _D
            cp_r = pltpu.async_copy(tv_hbm.at[r], row_v, sem_r)
            cp_x = pltpu.async_copy(x_hbm.at[pl.ds(f * _B, _B)], buf_v, sem_x)
            cp_r.wait()
            cp_x.wait()

            def gather_body(i, c2):
                base = i * (_VEC * _UNROLL)
                for u in range(_UNROLL):
                    sl = pl.ds(base + u * _VEC, _VEC)
                    xi = plsc.bitcast(buf_v[sl], jnp.int32)
                    buf_v[sl] = plsc.load_gather(row_v, [xi])
                return c2

            lax.fori_loop(0, _NITER, gather_body, 0)
            pltpu.sync_copy(buf_v, out_hbm.at[r])
            return carry

        lax.fori_loop(0, rpw, row_body, 0)

    return emb


_CHUNKS = (416, 416)
_KERNELS = []
_row = 0
for _n in _CHUNKS:
    _KERNELS.append((_row, _n, _make_kernel(_row, _n)))
    _row += _n


def kernel(X, tables):
    tv = jnp.transpose(tables, (0, 2, 1)).reshape(_R, _V)
    xt = jnp.transpose(X.astype(jnp.int32), (1, 0)).reshape(_F * _B)
    xf = lax.bitcast_convert_type(xt, jnp.float32)
    outs = [k(tv[r0:r0 + n], xf) for r0, n, k in _KERNELS]
    out = jnp.concatenate(outs, axis=0)
    return jnp.transpose(out.reshape(_F, _D, _B), (2, 0, 1))


# speedup vs baseline: 1.0288x; 1.0288x over previous
import functools

import jax
import jax.numpy as jnp
from jax import lax
from jax.experimental import pallas as pl
from jax.experimental.pallas import tpu as pltpu
from jax.experimental.pallas import tpu_sc as plsc

_F = 26
_V = 100000
_D = 32
_B = 16384

_NC = 2
_NS = 16
_NW = _NC * _NS

_R = _F * _D
_RPW = _R // _NW
_VEC = 16
_UNROLL = 8
_NITER = _B // (_VEC * _UNROLL)


def _make_kernel(row0, nrows):
    rpw = nrows // _NW
    mesh = plsc.VectorSubcoreMesh(core_axis_name="c", subcore_axis_name="s")

    @functools.partial(
        pl.kernel,
        mesh=mesh,
        out_type=jax.ShapeDtypeStruct((nrows, _B), jnp.float32),
        compiler_params=pltpu.CompilerParams(
            use_tc_tiling_on_sc=False, needs_layout_passes=False),
        scratch_types=[
            pltpu.VMEM((_V,), jnp.float32),
            pltpu.VMEM((_B,), jnp.float32),
            pltpu.SemaphoreType.DMA,
            pltpu.SemaphoreType.DMA,
        ],
    )
    def emb(tv_hbm, x_hbm, out_hbm, row_v, buf_v, sem_r, sem_x):
        wid = lax.axis_index("s") * _NC + lax.axis_index("c")
        r0 = wid * rpw

        def row_body(rl, carry):
            r = r0 + rl
            f = (row0 + r) // _D
            cp_r = pltpu.async_copy(tv_hbm.at[r], row_v, sem_r)
            cp_x = pltpu.async_copy(x_hbm.at[pl.ds(f * _B, _B)], buf_v, sem_x)
            cp_r.wait()
            cp_x.wait()

            def gather_body(i, c2):
                base = i * (_VEC * _UNROLL)
                for u in range(_UNROLL):
                    sl = pl.ds(base + u * _VEC, _VEC)
                    xi = plsc.bitcast(buf_v[sl], jnp.int32)
                    buf_v[sl] = plsc.load_gather(row_v, [xi])
                return c2

            lax.fori_loop(0, _NITER, gather_body, 0)
            pltpu.sync_copy(buf_v, out_hbm.at[r])
            return carry

        lax.fori_loop(0, rpw, row_body, 0)

    return emb


_CHUNKS = (416, 416)
_KERNELS = []
_row = 0
for _n in _CHUNKS:
    _KERNELS.append((_row, _n, _make_kernel(_row, _n)))
    _row += _n


def kernel(X, tables):
    xt = jnp.transpose(X.astype(jnp.int32), (1, 0)).reshape(_F * _B)
    xf = lax.bitcast_convert_type(xt, jnp.float32)
    outs = []
    for r0, n, k in _KERNELS:
        f0, f1 = r0 // _D, (r0 + n) // _D
        tvc = jnp.transpose(tables[f0:f1], (0, 2, 1)).reshape(n, _V)
        outs.append(k(tvc, xf))
    out = jnp.concatenate(outs, axis=0)
    return jnp.transpose(out.reshape(_F, _D, _B), (2, 0, 1))


# speedup vs baseline: 1.1800x; 1.1469x over previous
import functools

import jax
import jax.numpy as jnp
from jax import lax
from jax.experimental import pallas as pl
from jax.experimental.pallas import tpu as pltpu
from jax.experimental.pallas import tpu_sc as plsc

_F = 26
_V = 100000
_D = 32
_B = 16384

_NC = 2
_NS = 16
_NW = _NC * _NS

_R = _F * _D
_RPW = _R // _NW
_VEC = 16
_UNROLL = 8
_NITER = _B // (_VEC * _UNROLL)


def _make_kernel():
    mesh = plsc.VectorSubcoreMesh(core_axis_name="c", subcore_axis_name="s")

    @functools.partial(
        pl.kernel,
        mesh=mesh,
        out_type=jax.ShapeDtypeStruct((_R, _B), jnp.float32),
        compiler_params=pltpu.CompilerParams(
            use_tc_tiling_on_sc=False, needs_layout_passes=False),
        scratch_types=[
            pltpu.VMEM((_V,), jnp.float32),
            pltpu.VMEM((_B,), jnp.float32),
            pltpu.SemaphoreType.DMA,
            pltpu.SemaphoreType.DMA,
        ],
    )
    def emb(tv_hbm, x_hbm, out_hbm, row_v, buf_v, sem_r, sem_x):
        wid = lax.axis_index("s") * _NC + lax.axis_index("c")
        r0 = wid * _RPW

        def row_body(rl, carry):
            r = r0 + rl
            f = r // _D
            cp_r = pltpu.async_copy(tv_hbm.at[r], row_v, sem_r)
            cp_x = pltpu.async_copy(x_hbm.at[pl.ds(f * _B, _B)], buf_v, sem_x)
            cp_r.wait()
            cp_x.wait()

            def gather_body(i, c2):
                base = i * (_VEC * _UNROLL)
                for u in range(_UNROLL):
                    sl = pl.ds(base + u * _VEC, _VEC)
                    xi = plsc.bitcast(buf_v[sl], jnp.int32)
                    buf_v[sl] = plsc.load_gather(row_v, [xi])
                return c2

            lax.fori_loop(0, _NITER, gather_body, 0)
            pltpu.sync_copy(buf_v, out_hbm.at[r])
            return carry

        lax.fori_loop(0, _RPW, row_body, 0)

    return emb


_emb_kernel = _make_kernel()


def kernel(X, tables):
    tv = jnp.transpose(tables, (0, 2, 1)).reshape(_R, _V)
    xt = jnp.transpose(X.astype(jnp.int32), (1, 0)).reshape(_F * _B)
    xf = lax.bitcast_convert_type(xt, jnp.float32)
    out = _emb_kernel(tv, xf)
    return jnp.transpose(out.reshape(_F, _D, _B), (2, 0, 1))


# speedup vs baseline: 1.2198x; 1.0338x over previous
import functools

import jax
import jax.numpy as jnp
from jax import lax
from jax.experimental import pallas as pl
from jax.experimental.pallas import tpu as pltpu
from jax.experimental.pallas import tpu_sc as plsc

_F = 26
_V = 100000
_D = 32
_B = 16384

_NC = 2
_NS = 16
_NW = _NC * _NS

_R = _F * _D
_RPW = _R // _NW
_VEC = 16
_UNROLL = 8
_NITER = _B // (_VEC * _UNROLL)


def _make_kernel():
    mesh = plsc.VectorSubcoreMesh(core_axis_name="c", subcore_axis_name="s")

    @functools.partial(
        pl.kernel,
        mesh=mesh,
        out_type=jax.ShapeDtypeStruct((_R, _B), jnp.float32),
        compiler_params=pltpu.CompilerParams(
            use_tc_tiling_on_sc=False, needs_layout_passes=False),
        scratch_types=[
            pltpu.VMEM((_V,), jnp.float32),
            pltpu.VMEM((_B,), jnp.float32),
            pltpu.VMEM((_B // 2,), jnp.float32),
            pltpu.SemaphoreType.DMA,
        ],
    )
    def emb(tv_hbm, x_hbm, out_hbm, row_v, x_v, o_v, sem_r):
        wid = lax.axis_index("s") * _NC + lax.axis_index("c")
        r0 = wid * _RPW

        def row_body(rl, f_prev):
            r = r0 + rl
            f = r // _D
            cp_r = pltpu.async_copy(tv_hbm.at[r], row_v, sem_r)

            @pl.when(f != f_prev)
            def _():
                pltpu.sync_copy(x_hbm.at[pl.ds(f * _B, _B)], x_v)

            cp_r.wait()

            for h in range(2):
                def gather_body(i, c2, h=h):
                    base = h * (_B // 2) + i * (_VEC * _UNROLL)
                    obase = i * (_VEC * _UNROLL)
                    for u in range(_UNROLL):
                        sl = pl.ds(base + u * _VEC, _VEC)
                        osl = pl.ds(obase + u * _VEC, _VEC)
                        xi = plsc.bitcast(x_v[sl], jnp.int32)
                        o_v[osl] = plsc.load_gather(row_v, [xi])
                    return c2

                lax.fori_loop(0, _NITER // 2, gather_body, 0)
                pltpu.sync_copy(
                    o_v, out_hbm.at[r, pl.ds(h * (_B // 2), _B // 2)])
            return f

        lax.fori_loop(0, _RPW, row_body, -1)

    return emb


_emb_kernel = _make_kernel()


def kernel(X, tables):
    tv = jnp.transpose(tables, (0, 2, 1)).reshape(_R, _V)
    xt = jnp.transpose(X.astype(jnp.int32), (1, 0)).reshape(_F * _B)
    xf = lax.bitcast_convert_type(xt, jnp.float32)
    out = _emb_kernel(tv, xf)
    return jnp.transpose(out.reshape(_F, _D, _B), (2, 0, 1))
